# baseline (speedup 1.0000x reference)
import functools

import jax
import jax.numpy as jnp
from jax.experimental import pallas as pl
from jax.experimental.pallas import tpu as pltpu

_SPLIT = 8


def _eca_compute(x, w_ref, *, ntaps):
    hw = x.shape[-1]
    mean = jnp.sum(x, axis=-1, keepdims=True, dtype=jnp.float32) * (1.0 / hw)
    pad = ntaps // 2
    acc = mean * w_ref[pad]
    for t in range(ntaps):
        d = t - pad
        if d == 0:
            continue
        if d > 0:
            shifted = jnp.concatenate(
                [mean[d:, :], jnp.zeros((d, 1), jnp.float32)], axis=0)
        else:
            shifted = jnp.concatenate(
                [jnp.zeros((-d, 1), jnp.float32), mean[:d, :]], axis=0)
        acc = acc + shifted * w_ref[t]
    return x * jax.nn.sigmoid(acc)


def _pipe_body(w_ref, x_hbm, o_hbm, x_buf, o_buf, in_sem, out_sem,
               *, ntaps, n_steps, rows):
    sub = rows // _SPLIT

    def dma_in(slot, step):
        for k in range(_SPLIT):
            pltpu.make_async_copy(
                x_hbm.at[step, pl.ds(k * sub, sub), :],
                x_buf.at[slot, pl.ds(k * sub, sub), :],
                in_sem.at[slot]).start()

    def wait_in(slot):
        pltpu.make_async_copy(x_hbm.at[0], x_buf.at[slot],
                              in_sem.at[slot]).wait()

    def dma_out(slot, step):
        for k in range(_SPLIT):
            pltpu.make_async_copy(
                o_buf.at[slot, pl.ds(k * sub, sub), :],
                o_hbm.at[step, pl.ds(k * sub, sub), :],
                out_sem.at[slot]).start(priority=1)

    def wait_out(slot):
        pltpu.make_async_copy(o_buf.at[slot], o_hbm.at[0],
                              out_sem.at[slot]).wait()

    dma_in(0, 0)

    def body(step, _):
        cur = jax.lax.rem(step, 2)
        nxt = jax.lax.rem(step + 1, 2)

        @pl.when(step + 1 < n_steps)
        def _():
            dma_in(nxt, step + 1)

        wait_in(cur)

        @pl.when(step >= 2)
        def _():
            wait_out(cur)

        o_buf[cur] = _eca_compute(x_buf[cur], w_ref, ntaps=ntaps)
        dma_out(cur, step)
        return ()

    jax.lax.fori_loop(0, n_steps, body, (), unroll=False)
    wait_out(jax.lax.rem(n_steps - 2, 2))
    wait_out(jax.lax.rem(n_steps - 1, 2))


def kernel(x_nchw, conv_weight):
    B, C, H, W = x_nchw.shape
    HW = H * W
    K = conv_weight.shape[0]
    x = x_nchw.reshape(B, C, HW)

    out = pl.pallas_call(
        functools.partial(_pipe_body, ntaps=K, n_steps=B, rows=C),
        out_shape=jax.ShapeDtypeStruct((B, C, HW), x.dtype),
        in_specs=[
            pl.BlockSpec(memory_space=pltpu.SMEM),
            pl.BlockSpec(memory_space=pl.ANY),
        ],
        out_specs=pl.BlockSpec(memory_space=pl.ANY),
        scratch_shapes=[
            pltpu.VMEM((2, C, HW), x.dtype),
            pltpu.VMEM((2, C, HW), x.dtype),
            pltpu.SemaphoreType.DMA((2,)),
            pltpu.SemaphoreType.DMA((2,)),
        ],
        compiler_params=pltpu.CompilerParams(
            vmem_limit_bytes=40 * 1024 * 1024,
        ),
    )(conv_weight.astype(jnp.float32), x)

    return out.reshape(B, C, H, W)
